# Initial kernel scaffold; baseline (speedup 1.0000x reference)
import jax
import jax.numpy as jnp
from jax.experimental import pallas as pl


def kernel(x, edge_index, weight, bias):
    raise NotImplementedError("write your pallas kernel here")


# speedup vs baseline: 6.5383x; 6.5383x over previous
import functools

import jax
import jax.numpy as jnp
from jax import lax
from jax.experimental import pallas as pl
from jax.experimental.pallas import tpu as pltpu
from jax.experimental.pallas import tpu_sc as plsc

N_NODES = 10000
N_EDGES = 160000
D = 256
H = D // 2
NS = 16
EPT = N_EDGES // NS
B = 80
KB = EPT // B
NPAD = 10240
RPT = NPAD // NS


def _sc_segment_sum(x2, col2, rowt, zer):
    mesh = plsc.VectorSubcoreMesh(core_axis_name="c", subcore_axis_name="s")

    @functools.partial(
        pl.kernel,
        out_type=jax.ShapeDtypeStruct((2, NPAD, H), jnp.float32),
        mesh=mesh,
        scratch_types=[
            pltpu.VMEM((KB, B), jnp.int32),
            pltpu.VMEM((KB, B), jnp.int32),
            pltpu.VMEM((B, H), jnp.float32),
            pltpu.VMEM_SHARED((NPAD, H), jnp.float32),
            pltpu.SemaphoreType.DMA,
        ],
    )
    def sc_accum(x2_hbm, col2_hbm, rowt_hbm, zer_hbm, out_hbm,
                 ci, ri, buf, s_sh, sem):
        c = lax.axis_index("c")
        t = lax.axis_index("s")
        pltpu.sync_copy(zer_hbm, s_sh.at[pl.ds(t * RPT, RPT)])
        pltpu.sync_copy(col2_hbm.at[c, t], ci)
        pltpu.sync_copy(rowt_hbm.at[t], ri)
        plsc.subcore_barrier()

        def step(k, carry):
            pltpu.async_copy(x2_hbm.at[ci.at[k]], buf, sem).wait()
            pltpu.sync_copy(buf, s_sh.at[ri.at[k]], add=True)
            return carry

        lax.fori_loop(0, KB, step, 0)
        plsc.subcore_barrier()
        pltpu.sync_copy(s_sh.at[pl.ds(t * RPT, RPT)],
                        out_hbm.at[c, pl.ds(t * RPT, RPT)])

    return sc_accum(x2, col2, rowt, zer)


def _tc_finish(x, s0, s1, wt, bias2):
    blk = 2000
    grid = (N_NODES // blk,)

    def body(x_ref, s0_ref, s1_ref, wt_ref, b_ref, o_ref):
        xs = x_ref[...] * jnp.concatenate([s0_ref[...], s1_ref[...]], axis=-1)
        o_ref[...] = (jnp.dot(xs, wt_ref[...],
                              preferred_element_type=jnp.float32)
                      + b_ref[...])

    return pl.pallas_call(
        body,
        grid=grid,
        in_specs=[
            pl.BlockSpec((blk, D), lambda i: (i, 0)),
            pl.BlockSpec((blk, H), lambda i: (i, 0)),
            pl.BlockSpec((blk, H), lambda i: (i, 0)),
            pl.BlockSpec((D, D), lambda i: (0, 0)),
            pl.BlockSpec((1, D), lambda i: (0, 0)),
        ],
        out_specs=pl.BlockSpec((blk, D), lambda i: (i, 0)),
        out_shape=jax.ShapeDtypeStruct((N_NODES, D), jnp.float32),
    )(x, s0, s1, wt, bias2)


def kernel(x, edge_index, weight, bias):
    row = edge_index[0].astype(jnp.int32)
    col = edge_index[1].astype(jnp.int32)
    x2 = x.reshape(2 * N_NODES, H)
    col2 = jnp.stack([col * 2, col * 2 + 1]).reshape(2, NS, KB, B)
    rowt = row.reshape(NS, KB, B)
    zer = jnp.zeros((RPT, H), dtype=jnp.float32)

    s = _sc_segment_sum(x2, col2, rowt, zer)

    wt = weight.T
    bias2 = bias[None, :]
    return _tc_finish(x, s[0], s[1], wt, bias2)


# speedup vs baseline: 7.0377x; 1.0764x over previous
import functools

import jax
import jax.numpy as jnp
from jax import lax
from jax.experimental import pallas as pl
from jax.experimental.pallas import tpu as pltpu
from jax.experimental.pallas import tpu_sc as plsc

N_NODES = 10000
N_EDGES = 160000
D = 256
H = D // 2
NS = 16
EPT = N_EDGES // NS
B = 128
KR = 79
KB = KR + 1
NPAD = 10240
RPT = NPAD // NS


def _sc_segment_sum(x2, idx_all, zer):
    mesh = plsc.VectorSubcoreMesh(core_axis_name="c", subcore_axis_name="s")

    @functools.partial(
        pl.kernel,
        out_type=jax.ShapeDtypeStruct((2, NPAD, H), jnp.float32),
        mesh=mesh,
        scratch_types=[
            pltpu.VMEM((2, B), jnp.int32),
            pltpu.VMEM((2, B), jnp.int32),
            pltpu.VMEM((B, H), jnp.float32),
            pltpu.VMEM((B, H), jnp.float32),
            pltpu.VMEM_SHARED((NPAD, H), jnp.float32),
            pltpu.SemaphoreType.DMA,
            pltpu.SemaphoreType.DMA,
            pltpu.SemaphoreType.DMA,
            pltpu.SemaphoreType.DMA,
        ],
    )
    def sc_accum(x2_hbm, idx_hbm, zer_hbm, out_hbm,
                 ib0, ib1, buf0, buf1, s_sh, si0, si1, sg0, sg1):
        c = lax.axis_index("c")
        t = lax.axis_index("s")
        pltpu.sync_copy(zer_hbm, s_sh.at[pl.ds(t * RPT, RPT)])
        plsc.subcore_barrier()

        pltpu.sync_copy(idx_hbm.at[c, t, 0], ib0)
        pltpu.async_copy(x2_hbm.at[ib0.at[0]], buf0, sg0)
        pltpu.async_copy(idx_hbm.at[c, t, 1], ib1, si1)

        def half_step(k, ib_a, si_a, buf_a, sg_a, ib_b, si_b, buf_b, sg_b):
            pltpu.make_async_copy(idx_hbm.at[c, t, k + 1], ib_b, si_b).wait()
            pltpu.async_copy(x2_hbm.at[ib_b.at[0]], buf_b, sg_b)
            pltpu.make_async_copy(x2_hbm.at[ib_a.at[0]], buf_a, sg_a).wait()
            pltpu.sync_copy(buf_a, s_sh.at[ib_a.at[1]], add=True)
            pltpu.async_copy(idx_hbm.at[c, t, k + 2], ib_a, si_a)

        def step(j, carry):
            k0 = 2 * j
            half_step(k0, ib0, si0, buf0, sg0, ib1, si1, buf1, sg1)
            half_step(k0 + 1, ib1, si1, buf1, sg1, ib0, si0, buf0, sg0)
            return carry

        lax.fori_loop(0, (KR - 1) // 2, step, 0)
        pltpu.make_async_copy(x2_hbm.at[ib0.at[0]], buf0, sg0).wait()
        pltpu.sync_copy(buf0, s_sh.at[ib0.at[1]], add=True)
        pltpu.make_async_copy(idx_hbm.at[c, t, KR], ib1, si1).wait()
        plsc.subcore_barrier()
        pltpu.sync_copy(s_sh.at[pl.ds(t * RPT, RPT)],
                        out_hbm.at[c, pl.ds(t * RPT, RPT)])

    return sc_accum(x2, idx_all, zer)


def _tc_finish(x, s0, s1, wt, bias2):
    blk = 2000
    grid = (N_NODES // blk,)

    def body(x_ref, s0_ref, s1_ref, wt_ref, b_ref, o_ref):
        xs = x_ref[...] * jnp.concatenate([s0_ref[...], s1_ref[...]], axis=-1)
        o_ref[...] = (jnp.dot(xs, wt_ref[...],
                              preferred_element_type=jnp.float32)
                      + b_ref[...])

    return pl.pallas_call(
        body,
        grid=grid,
        in_specs=[
            pl.BlockSpec((blk, D), lambda i: (i, 0)),
            pl.BlockSpec((blk, H), lambda i: (i, 0)),
            pl.BlockSpec((blk, H), lambda i: (i, 0)),
            pl.BlockSpec((D, D), lambda i: (0, 0)),
            pl.BlockSpec((1, D), lambda i: (0, 0)),
        ],
        out_specs=pl.BlockSpec((blk, D), lambda i: (i, 0)),
        out_shape=jax.ShapeDtypeStruct((N_NODES, D), jnp.float32),
    )(x, s0, s1, wt, bias2)


def kernel(x, edge_index, weight, bias):
    row = edge_index[0].astype(jnp.int32)
    col = edge_index[1].astype(jnp.int32)
    x2 = x.reshape(2 * N_NODES, H)
    npad = KB * B - EPT
    colp = jnp.concatenate(
        [col.reshape(NS, EPT),
         jnp.zeros((NS, npad), jnp.int32)], axis=1)
    rowp = jnp.concatenate(
        [row.reshape(NS, EPT),
         jnp.full((NS, npad), NPAD - 1, jnp.int32)], axis=1)
    gidx = jnp.stack([colp * 2, colp * 2 + 1])
    sidx = jnp.broadcast_to(rowp, (2, NS, KB * B))
    idx_all = jnp.stack(
        [gidx.reshape(2, NS, KB, B), sidx.reshape(2, NS, KB, B)],
        axis=3)
    zer = jnp.zeros((RPT, H), dtype=jnp.float32)

    s = _sc_segment_sum(x2, idx_all, zer)

    wt = weight.T
    bias2 = bias[None, :]
    return _tc_finish(x, s[0], s[1], wt, bias2)
